# baseline (speedup 1.0000x reference)
import functools

import jax
import jax.numpy as jnp
from jax import lax
from jax.experimental import pallas as pl
from jax.experimental.pallas import tpu as pltpu
from jax.experimental.pallas import tpu_sc as plsc

_N = 10000
_E = 320000
_D = 128
_H = 64

_NC = 2
_NS = 16
_NW = _NC * _NS
_NPAD = 10240
_ROWS_PER_SUB = _NPAD // _NS
_CHUNK = 128
_INNER = 8
_OUTER0 = 20
_OUTER1 = 0
_CPW0 = _INNER * _OUTER0
_CPW1 = _INNER * _OUTER1
_NCHUNKS = _NS * (_CPW0 + _CPW1)
_EPAD = _NCHUNKS * _CHUNK


_HALF = _INNER // 2


def _sc_pass_body(with_cnt, stage_y, *refs):
    if with_cnt:
        (y, srcm, dstm, za, zc, ones_h, agg_out, cnt_out,
         agg_sh, cnt_sh, *rest) = refs
    else:
        (y, srcm, dstm, za, agg_out, agg_sh, *rest) = refs
    if stage_y:
        y_sh, *rest = rest
    if with_cnt:
        (src_v, dst_v, buf_a, buf_b, ones_v,
         sem_g, sem_sa, sem_sb, sem_o) = rest
    else:
        (src_v, dst_v, buf_a, buf_b,
         sem_g, sem_sa, sem_sb, sem_o) = rest
    cid = lax.axis_index("c")
    sid = lax.axis_index("s")
    row0 = sid * _ROWS_PER_SUB
    stripe = pl.ds(row0, _ROWS_PER_SUB)
    y_src = y_sh if stage_y else y
    if stage_y:
        pltpu.sync_copy(y.at[stripe], y_sh.at[stripe])
    pltpu.sync_copy(za, agg_sh.at[stripe])
    if with_cnt:
        pltpu.sync_copy(zc, cnt_sh.at[stripe])
        pltpu.sync_copy(ones_h, ones_v)
    plsc.subcore_barrier()

    idx_row0 = jnp.where(cid == 0, sid * _CPW0, 0)
    outer = jnp.where(cid == 0, _OUTER0, _OUTER1)

    def load_idx(j, p):
        base = idx_row0 + j * _INNER
        pltpu.sync_copy(srcm.at[pl.ds(base, _INNER)], src_v.at[p])
        pltpu.sync_copy(dstm.at[pl.ds(base, _INNER)], dst_v.at[p])

    def fire_gathers(p, t0, buf):
        return [
            pltpu.async_copy(
                y_src.at[src_v.at[p, t0 + t]],
                buf.at[pl.ds(t * _CHUNK, _CHUNK)], sem_g)
            for t in range(_HALF)
        ]

    def fire_scatters(p, t0, buf, sem):
        ds = []
        for t in range(_HALF):
            ds.append(pltpu.async_copy(
                buf.at[pl.ds(t * _CHUNK, _CHUNK)],
                agg_sh.at[dst_v.at[p, t0 + t]], sem, add=True))
            if with_cnt:
                ds.append(pltpu.async_copy(
                    ones_v, cnt_sh.at[dst_v.at[p, t0 + t]], sem_o, add=True))
        return ds

    def drain(buf, sem):
        for t in range(_HALF):
            pltpu.make_async_copy(
                buf.at[pl.ds(t * _CHUNK, _CHUNK)],
                agg_sh.at[pl.ds(0, _CHUNK)], sem).wait()

    def drain_ones():
        for _ in range(_INNER):
            pltpu.make_async_copy(
                ones_v, cnt_sh.at[pl.ds(0, _CHUNK)], sem_o).wait()

    @pl.when(outer != 0)
    def _():
        load_idx(0, 0)

    @pl.loop(0, outer)
    def _(j):
        p = j % 2

        @pl.when(j != 0)
        def _():
            drain(buf_a, sem_sa)

        gA = fire_gathers(p, 0, buf_a)

        @pl.when(j != 0)
        def _():
            drain(buf_b, sem_sb)
            if with_cnt:
                drain_ones()

        for d in gA:
            d.wait()
        fire_scatters(p, 0, buf_a, sem_sa)
        gB = fire_gathers(p, _HALF, buf_b)

        @pl.when(j != outer - 1)
        def _():
            load_idx(j + 1, 1 - p)

        for d in gB:
            d.wait()
        fire_scatters(p, _HALF, buf_b, sem_sb)

    @pl.when(outer != 0)
    def _():
        drain(buf_a, sem_sa)
        drain(buf_b, sem_sb)
        if with_cnt:
            drain_ones()

    plsc.subcore_barrier()
    sl = pl.ds(row0, _ROWS_PER_SUB)
    pltpu.sync_copy(agg_sh.at[sl], agg_out.at[cid, sl])
    if with_cnt:
        pltpu.sync_copy(cnt_sh.at[sl], cnt_out.at[cid, sl])


def _make_sc_pass(with_cnt, stage_y):
    mesh = plsc.VectorSubcoreMesh(core_axis_name="c", subcore_axis_name="s")
    out_type = [jax.ShapeDtypeStruct((_NC, _NPAD, _H), jnp.float32)]
    scratch = [
        pltpu.VMEM_SHARED((_NPAD, _H), jnp.float32),
    ]
    if stage_y:
        scratch.append(pltpu.VMEM_SHARED((_NPAD, _H), jnp.float32))
    scratch += [
        pltpu.VMEM((2, _INNER, _CHUNK), jnp.int32),
        pltpu.VMEM((2, _INNER, _CHUNK), jnp.int32),
        pltpu.VMEM((_HALF * _CHUNK, _H), jnp.float32),
        pltpu.VMEM((_HALF * _CHUNK, _H), jnp.float32),
    ]
    if with_cnt:
        out_type.append(jax.ShapeDtypeStruct((_NC, _NPAD, 16), jnp.float32))
        scratch.insert(1, pltpu.VMEM_SHARED((_NPAD, 16), jnp.float32))
        scratch.append(pltpu.VMEM((_CHUNK, 16), jnp.float32))
    scratch += [pltpu.SemaphoreType.DMA] * 4
    return pl.kernel(
        functools.partial(_sc_pass_body, with_cnt, stage_y),
        out_type=out_type,
        mesh=mesh,
        scratch_types=scratch,
        compiler_params=pltpu.CompilerParams(use_tc_tiling_on_sc=False),
        name="sage_edge_pass_cnt" if with_cnt else "sage_edge_pass",
    )


_sc_pass_cnt = _make_sc_pass(True, False)
_sc_pass = _make_sc_pass(False, False)

_BM = 1024


def _tc_a_body(x_ref, wl_ref, wr_ref, b_ref, y_ref, z_ref):
    x = x_ref[...]
    y_ref[...] = jnp.dot(x, wl_ref[...], preferred_element_type=jnp.float32)
    z_ref[...] = (
        jnp.dot(x, wr_ref[...], preferred_element_type=jnp.float32) + b_ref[...]
    )


def _tc_b_body(a0, a1, c0, c1, z1, wl, wr, b, y2, z2):
    cnt = c0[...][:, :1] + c1[...][:, :1]
    mean = (a0[...] + a1[...]) / jnp.maximum(cnt, 1.0)
    h = jnp.maximum(mean + z1[...], 0.0)
    y2[...] = jnp.dot(h, wl[...], preferred_element_type=jnp.float32)
    z2[...] = jnp.dot(h, wr[...], preferred_element_type=jnp.float32) + b[...]


def _tc_c_body(a0, a1, c0, c1, z2, wo, b, o):
    cnt = c0[...][:, :1] + c1[...][:, :1]
    mean = (a0[...] + a1[...]) / jnp.maximum(cnt, 1.0)
    h = jnp.maximum(mean + z2[...], 0.0)
    o[...] = jnp.sum(h * wo[...], axis=1) + b[0, 0]


def _rows_spec(w):
    return pl.BlockSpec((_BM, w), lambda i: (i, 0))


def _full_spec(shape):
    return pl.BlockSpec(shape, lambda i: tuple(0 for _ in shape))


_GRID = (_NPAD // _BM,)

_tc_a = pl.pallas_call(
    _tc_a_body,
    grid=_GRID,
    in_specs=[_rows_spec(_D), _full_spec((_D, _H)), _full_spec((_D, _H)),
              _full_spec((1, _H))],
    out_specs=[_rows_spec(_H), _rows_spec(_H)],
    out_shape=[jax.ShapeDtypeStruct((_NPAD, _H), jnp.float32)] * 2,
)

_tc_b = pl.pallas_call(
    _tc_b_body,
    grid=_GRID,
    in_specs=[_rows_spec(_H), _rows_spec(_H), _rows_spec(16), _rows_spec(16),
              _rows_spec(_H), _full_spec((_H, _H)), _full_spec((_H, _H)),
              _full_spec((1, _H))],
    out_specs=[_rows_spec(_H), _rows_spec(_H)],
    out_shape=[jax.ShapeDtypeStruct((_NPAD, _H), jnp.float32)] * 2,
)

_tc_c = pl.pallas_call(
    _tc_c_body,
    grid=_GRID,
    in_specs=[_rows_spec(_H), _rows_spec(_H), _rows_spec(16), _rows_spec(16),
              _rows_spec(_H), _full_spec((1, _H)), _full_spec((1, 1))],
    out_specs=pl.BlockSpec((_BM,), lambda i: (i,)),
    out_shape=jax.ShapeDtypeStruct((_NPAD,), jnp.float32),
)


def kernel(x, edge_index, W1l, b1l, W1r, W2l, b2l, W2r, Wo, bo):
    xp = jnp.zeros((_NPAD, _D), jnp.float32).at[:_N].set(x)
    pad = jnp.full((_EPAD - _E,), _NPAD - 1, jnp.int32)
    srcm = jnp.concatenate([edge_index[0], pad]).reshape(_EPAD // _CHUNK, _CHUNK)
    dstm = jnp.concatenate([edge_index[1], pad]).reshape(_EPAD // _CHUNK, _CHUNK)
    za = jnp.zeros((_ROWS_PER_SUB, _H), jnp.float32)
    zc = jnp.zeros((_ROWS_PER_SUB, 16), jnp.float32)
    ones_h = jnp.ones((_CHUNK, 16), jnp.float32)

    y1, z1 = _tc_a(xp, W1l, W1r, b1l.reshape(1, _H))
    agg1, cnt = _sc_pass_cnt(y1, srcm, dstm, za, zc, ones_h)
    y2, z2 = _tc_b(agg1[0], agg1[1], cnt[0], cnt[1], z1,
                   W2l, W2r, b2l.reshape(1, _H))
    (agg2,) = _sc_pass(y2, srcm, dstm, za)
    out = _tc_c(agg2[0], agg2[1], cnt[0], cnt[1], z2,
                Wo.reshape(1, _H), bo.reshape(1, 1))
    return out[:_N]


# speedup vs baseline: 1.7794x; 1.7794x over previous
import functools

import jax
import jax.numpy as jnp
from jax import lax
from jax.experimental import pallas as pl
from jax.experimental.pallas import tpu as pltpu
from jax.experimental.pallas import tpu_sc as plsc

_N = 10000
_E = 320000
_D = 128
_H = 64

_NC = 2
_NS = 16
_NW = _NC * _NS
_NPAD = 10240
_ROWS_PER_SUB = _NPAD // _NS
_CHUNK = 128
_INNER = 8
_OUTER0 = 19
_OUTER1 = 1
_CPW0 = _INNER * _OUTER0
_CPW1 = _INNER * _OUTER1
_NCHUNKS = _NS * (_CPW0 + _CPW1)
_EPAD = _NCHUNKS * _CHUNK


_HALF = _INNER // 2


def _sc_pass_body(with_cnt, stage_y, *refs):
    if with_cnt:
        (y, srcm, dstm, za, zc, ones_h, agg_out, cnt_out,
         agg_sh, cnt_sh, *rest) = refs
    else:
        (y, srcm, dstm, za, agg_out, agg_sh, *rest) = refs
    if stage_y:
        y_sh, *rest = rest
    if with_cnt:
        (src_v, dst_v, buf_a, buf_b, ones_v,
         sem_g, sem_sa, sem_sb, sem_o) = rest
    else:
        (src_v, dst_v, buf_a, buf_b,
         sem_g, sem_sa, sem_sb, sem_o) = rest
    cid = lax.axis_index("c")
    sid = lax.axis_index("s")
    row0 = sid * _ROWS_PER_SUB
    stripe = pl.ds(row0, _ROWS_PER_SUB)
    y_src = y_sh if stage_y else y
    if stage_y:
        pltpu.sync_copy(y.at[stripe], y_sh.at[stripe])
    pltpu.sync_copy(za, agg_sh.at[stripe])
    if with_cnt:
        pltpu.sync_copy(zc, cnt_sh.at[stripe])
        pltpu.sync_copy(ones_h, ones_v)
    plsc.subcore_barrier()

    idx_row0 = jnp.where(cid == 0, sid * _CPW0, _NS * _CPW0 + sid * _CPW1)
    outer = jnp.where(cid == 0, _OUTER0, _OUTER1)

    def load_idx(j, p):
        base = idx_row0 + j * _INNER
        pltpu.sync_copy(srcm.at[pl.ds(base, _INNER)], src_v.at[p])
        pltpu.sync_copy(dstm.at[pl.ds(base, _INNER)], dst_v.at[p])

    def fire_gathers(p, t0, buf):
        return [
            pltpu.async_copy(
                y_src.at[src_v.at[p, t0 + t]],
                buf.at[pl.ds(t * _CHUNK, _CHUNK)], sem_g)
            for t in range(_HALF)
        ]

    def fire_scatters(p, t0, buf, sem):
        ds = []
        for t in range(_HALF):
            ds.append(pltpu.async_copy(
                buf.at[pl.ds(t * _CHUNK, _CHUNK)],
                agg_sh.at[dst_v.at[p, t0 + t]], sem, add=True))
            if with_cnt:
                ds.append(pltpu.async_copy(
                    ones_v, cnt_sh.at[dst_v.at[p, t0 + t]], sem_o, add=True))
        return ds

    def drain(buf, sem):
        for t in range(_HALF):
            pltpu.make_async_copy(
                buf.at[pl.ds(t * _CHUNK, _CHUNK)],
                agg_sh.at[pl.ds(0, _CHUNK)], sem).wait()

    def drain_ones():
        for _ in range(_INNER):
            pltpu.make_async_copy(
                ones_v, cnt_sh.at[pl.ds(0, _CHUNK)], sem_o).wait()

    load_idx(0, 0)

    @pl.loop(0, outer)
    def _(j):
        p = j % 2

        @pl.when(j != 0)
        def _():
            drain(buf_a, sem_sa)

        gA = fire_gathers(p, 0, buf_a)

        @pl.when(j != 0)
        def _():
            drain(buf_b, sem_sb)
            if with_cnt:
                drain_ones()

        for d in gA:
            d.wait()
        fire_scatters(p, 0, buf_a, sem_sa)
        gB = fire_gathers(p, _HALF, buf_b)

        @pl.when(j != outer - 1)
        def _():
            load_idx(j + 1, 1 - p)

        for d in gB:
            d.wait()
        fire_scatters(p, _HALF, buf_b, sem_sb)

    drain(buf_a, sem_sa)
    drain(buf_b, sem_sb)
    if with_cnt:
        drain_ones()
    plsc.subcore_barrier()
    sl = pl.ds(row0, _ROWS_PER_SUB)
    pltpu.sync_copy(agg_sh.at[sl], agg_out.at[cid, sl])
    if with_cnt:
        pltpu.sync_copy(cnt_sh.at[sl], cnt_out.at[cid, sl])


def _make_sc_pass(with_cnt, stage_y):
    mesh = plsc.VectorSubcoreMesh(core_axis_name="c", subcore_axis_name="s")
    out_type = [jax.ShapeDtypeStruct((_NC, _NPAD, _H), jnp.bfloat16)]
    scratch = [
        pltpu.VMEM_SHARED((_NPAD, _H), jnp.bfloat16),
    ]
    if stage_y:
        scratch.append(pltpu.VMEM_SHARED((_NPAD, _H), jnp.float32))
    scratch += [
        pltpu.VMEM((2, _INNER, _CHUNK), jnp.int32),
        pltpu.VMEM((2, _INNER, _CHUNK), jnp.int32),
        pltpu.VMEM((_HALF * _CHUNK, _H), jnp.bfloat16),
        pltpu.VMEM((_HALF * _CHUNK, _H), jnp.bfloat16),
    ]
    if with_cnt:
        out_type.append(jax.ShapeDtypeStruct((_NC, _NPAD, 16), jnp.float32))
        scratch.insert(1, pltpu.VMEM_SHARED((_NPAD, 16), jnp.float32))
        scratch.append(pltpu.VMEM((_CHUNK, 16), jnp.float32))
    scratch += [pltpu.SemaphoreType.DMA] * 4
    return pl.kernel(
        functools.partial(_sc_pass_body, with_cnt, stage_y),
        out_type=out_type,
        mesh=mesh,
        scratch_types=scratch,
        compiler_params=pltpu.CompilerParams(use_tc_tiling_on_sc=False),
        name="sage_edge_pass_cnt" if with_cnt else "sage_edge_pass",
    )


_sc_pass_cnt = _make_sc_pass(True, False)
_sc_pass = _make_sc_pass(False, False)

_BM = 1024


def _tc_a_body(x_ref, wl_ref, wr_ref, b_ref, y_ref, z_ref):
    x = x_ref[...]
    y_ref[...] = jnp.dot(
        x, wl_ref[...], preferred_element_type=jnp.float32
    ).astype(jnp.bfloat16)
    z_ref[...] = (
        jnp.dot(x, wr_ref[...], preferred_element_type=jnp.float32) + b_ref[...]
    )


def _tc_b_body(a0, a1, c0, c1, z1, wl, wr, b, y2, z2):
    cnt = c0[...][:, :1] + c1[...][:, :1]
    agg = a0[...].astype(jnp.float32) + a1[...].astype(jnp.float32)
    mean = agg / jnp.maximum(cnt, 1.0)
    h = jnp.maximum(mean + z1[...], 0.0)
    y2[...] = jnp.dot(
        h, wl[...], preferred_element_type=jnp.float32
    ).astype(jnp.bfloat16)
    z2[...] = jnp.dot(h, wr[...], preferred_element_type=jnp.float32) + b[...]


def _tc_c_body(a0, a1, c0, c1, z2, wo, b, o):
    cnt = c0[...][:, :1] + c1[...][:, :1]
    agg = a0[...].astype(jnp.float32) + a1[...].astype(jnp.float32)
    mean = agg / jnp.maximum(cnt, 1.0)
    h = jnp.maximum(mean + z2[...], 0.0)
    o[...] = jnp.sum(h * wo[...], axis=1) + b[0, 0]


def _rows_spec(w):
    return pl.BlockSpec((_BM, w), lambda i: (i, 0))


def _full_spec(shape):
    return pl.BlockSpec(shape, lambda i: tuple(0 for _ in shape))


_GRID = (_NPAD // _BM,)

_tc_a = pl.pallas_call(
    _tc_a_body,
    grid=_GRID,
    in_specs=[_rows_spec(_D), _full_spec((_D, _H)), _full_spec((_D, _H)),
              _full_spec((1, _H))],
    out_specs=[_rows_spec(_H), _rows_spec(_H)],
    out_shape=[jax.ShapeDtypeStruct((_NPAD, _H), jnp.bfloat16),
               jax.ShapeDtypeStruct((_NPAD, _H), jnp.float32)],
)

_tc_b = pl.pallas_call(
    _tc_b_body,
    grid=_GRID,
    in_specs=[_rows_spec(_H), _rows_spec(_H), _rows_spec(16), _rows_spec(16),
              _rows_spec(_H), _full_spec((_H, _H)), _full_spec((_H, _H)),
              _full_spec((1, _H))],
    out_specs=[_rows_spec(_H), _rows_spec(_H)],
    out_shape=[jax.ShapeDtypeStruct((_NPAD, _H), jnp.bfloat16),
               jax.ShapeDtypeStruct((_NPAD, _H), jnp.float32)],
)

_tc_c = pl.pallas_call(
    _tc_c_body,
    grid=_GRID,
    in_specs=[_rows_spec(_H), _rows_spec(_H), _rows_spec(16), _rows_spec(16),
              _rows_spec(_H), _full_spec((1, _H)), _full_spec((1, 1))],
    out_specs=pl.BlockSpec((_BM,), lambda i: (i,)),
    out_shape=jax.ShapeDtypeStruct((_NPAD,), jnp.float32),
)


def kernel(x, edge_index, W1l, b1l, W1r, W2l, b2l, W2r, Wo, bo):
    xp = jnp.zeros((_NPAD, _D), jnp.float32).at[:_N].set(x)
    pad = jnp.full((_EPAD - _E,), _NPAD - 1, jnp.int32)
    srcm = jnp.concatenate([edge_index[0], pad]).reshape(_EPAD // _CHUNK, _CHUNK)
    dstm = jnp.concatenate([edge_index[1], pad]).reshape(_EPAD // _CHUNK, _CHUNK)
    za = jnp.zeros((_ROWS_PER_SUB, _H), jnp.bfloat16)
    zc = jnp.zeros((_ROWS_PER_SUB, 16), jnp.float32)
    ones_h = jnp.ones((_CHUNK, 16), jnp.float32)

    y1, z1 = _tc_a(xp, W1l, W1r, b1l.reshape(1, _H))
    agg1, cnt = _sc_pass_cnt(y1, srcm, dstm, za, zc, ones_h)
    y2, z2 = _tc_b(agg1[0], agg1[1], cnt[0], cnt[1], z1,
                   W2l, W2r, b2l.reshape(1, _H))
    (agg2,) = _sc_pass(y2, srcm, dstm, za)
    out = _tc_c(agg2[0], agg2[1], cnt[0], cnt[1], z2,
                Wo.reshape(1, _H), bo.reshape(1, 1))
    return out[:_N]


# speedup vs baseline: 1.8390x; 1.0335x over previous
import functools

import jax
import jax.numpy as jnp
from jax import lax
from jax.experimental import pallas as pl
from jax.experimental.pallas import tpu as pltpu
from jax.experimental.pallas import tpu_sc as plsc

_N = 10000
_E = 320000
_D = 128
_H = 64

_NC = 2
_NS = 16
_NW = _NC * _NS
_NPAD = 10240
_ROWS_PER_SUB = _NPAD // _NS
_CHUNK = 128
_INNER = 8
_OUTER0 = 19
_OUTER1 = 1
_CPW0 = _INNER * _OUTER0
_CPW1 = _INNER * _OUTER1
_NCHUNKS = _NS * (_CPW0 + _CPW1)
_EPAD = _NCHUNKS * _CHUNK


_HALF = _INNER // 2


def _sc_pass_body(with_cnt, stage_y, *refs):
    if with_cnt:
        (y, srcm, dstm, za, zc, ones_h, agg_out, cnt_out,
         agg_sh, cnt_sh, *rest) = refs
    else:
        (y, srcm, dstm, za, agg_out, agg_sh, *rest) = refs
    if stage_y:
        y_sh, *rest = rest
    if with_cnt:
        (src_v, dst_v, buf_a, buf_b, ones_v,
         sem_g, sem_sa, sem_sb, sem_o) = rest
    else:
        (src_v, dst_v, buf_a, buf_b,
         sem_g, sem_sa, sem_sb, sem_o) = rest
    cid = lax.axis_index("c")
    sid = lax.axis_index("s")
    row0 = sid * _ROWS_PER_SUB
    stripe = pl.ds(row0, _ROWS_PER_SUB)
    y_src = y_sh if stage_y else y
    if stage_y:
        pltpu.sync_copy(y.at[stripe], y_sh.at[stripe])
    pltpu.sync_copy(za, agg_sh.at[stripe])
    if with_cnt:
        pltpu.sync_copy(zc, cnt_sh.at[stripe])
        pltpu.sync_copy(ones_h, ones_v)
    plsc.subcore_barrier()

    idx_row0 = jnp.where(cid == 0, sid * _CPW0, _NS * _CPW0 + sid * _CPW1)
    outer = jnp.where(cid == 0, _OUTER0, _OUTER1)

    def load_idx(j, p):
        base = idx_row0 + j * _INNER
        pltpu.sync_copy(srcm.at[pl.ds(base, _INNER)], src_v.at[p])
        pltpu.sync_copy(dstm.at[pl.ds(base, _INNER)], dst_v.at[p])

    def fire_gathers(p, t0, buf):
        return [
            pltpu.async_copy(
                y_src.at[src_v.at[p, t0 + t]],
                buf.at[pl.ds(t * _CHUNK, _CHUNK)], sem_g)
            for t in range(_HALF)
        ]

    def fire_scatters(p, t0, buf, sem):
        ds = []
        for t in range(_HALF):
            ds.append(pltpu.async_copy(
                buf.at[pl.ds(t * _CHUNK, _CHUNK)],
                agg_sh.at[dst_v.at[p, t0 + t]], sem, add=True))
            if with_cnt:
                ds.append(pltpu.async_copy(
                    ones_v, cnt_sh.at[dst_v.at[p, t0 + t]], sem_o, add=True))
        return ds

    def drain(buf, sem):
        for t in range(_HALF):
            pltpu.make_async_copy(
                buf.at[pl.ds(t * _CHUNK, _CHUNK)],
                agg_sh.at[pl.ds(0, _CHUNK)], sem).wait()

    def drain_ones():
        for _ in range(_INNER):
            pltpu.make_async_copy(
                ones_v, cnt_sh.at[pl.ds(0, _CHUNK)], sem_o).wait()

    load_idx(0, 0)

    @pl.loop(0, outer)
    def _(j):
        p = j % 2

        @pl.when(j != 0)
        def _():
            drain(buf_a, sem_sa)

        gA = fire_gathers(p, 0, buf_a)

        @pl.when(j != 0)
        def _():
            drain(buf_b, sem_sb)
            if with_cnt:
                drain_ones()

        for d in gA:
            d.wait()
        fire_scatters(p, 0, buf_a, sem_sa)
        gB = fire_gathers(p, _HALF, buf_b)

        @pl.when(j != outer - 1)
        def _():
            load_idx(j + 1, 1 - p)

        for d in gB:
            d.wait()
        fire_scatters(p, _HALF, buf_b, sem_sb)

    drain(buf_a, sem_sa)
    drain(buf_b, sem_sb)
    if with_cnt:
        drain_ones()
    plsc.subcore_barrier()
    sl = pl.ds(row0, _ROWS_PER_SUB)
    pltpu.sync_copy(agg_sh.at[sl], agg_out.at[cid, sl])
    if with_cnt:
        pltpu.sync_copy(cnt_sh.at[sl], cnt_out.at[cid, sl])


def _make_sc_pass(with_cnt, stage_y):
    mesh = plsc.VectorSubcoreMesh(core_axis_name="c", subcore_axis_name="s")
    out_type = [jax.ShapeDtypeStruct((_NC, _NPAD, _H), jnp.bfloat16)]
    scratch = [
        pltpu.VMEM_SHARED((_NPAD, _H), jnp.bfloat16),
    ]
    if stage_y:
        scratch.append(pltpu.VMEM_SHARED((_NPAD, _H), jnp.float32))
    scratch += [
        pltpu.VMEM((2, _INNER, _CHUNK), jnp.int32),
        pltpu.VMEM((2, _INNER, _CHUNK), jnp.int32),
        pltpu.VMEM((_HALF * _CHUNK, _H), jnp.bfloat16),
        pltpu.VMEM((_HALF * _CHUNK, _H), jnp.bfloat16),
    ]
    if with_cnt:
        out_type.append(jax.ShapeDtypeStruct((_NC, _NPAD, 16), jnp.float32))
        scratch.insert(1, pltpu.VMEM_SHARED((_NPAD, 16), jnp.float32))
        scratch.append(pltpu.VMEM((_CHUNK, 16), jnp.float32))
    scratch += [pltpu.SemaphoreType.DMA] * 4
    return pl.kernel(
        functools.partial(_sc_pass_body, with_cnt, stage_y),
        out_type=out_type,
        mesh=mesh,
        scratch_types=scratch,
        compiler_params=pltpu.CompilerParams(use_tc_tiling_on_sc=False),
        name="sage_edge_pass_cnt" if with_cnt else "sage_edge_pass",
    )


_sc_pass_cnt = _make_sc_pass(True, False)
_sc_pass = _make_sc_pass(False, False)

_BM = 1024


def _tc_a_body(x_ref, wl_ref, wr_ref, b_ref, y_ref, z_ref):
    x = x_ref[...].astype(jnp.bfloat16)
    y_ref[...] = jnp.dot(
        x, wl_ref[...].astype(jnp.bfloat16), preferred_element_type=jnp.float32
    ).astype(jnp.bfloat16)
    z_ref[...] = (
        jnp.dot(x, wr_ref[...].astype(jnp.bfloat16),
                preferred_element_type=jnp.float32) + b_ref[...]
    )


def _tc_b_body(a, c, z1, wl, wr, b, y2, z2):
    av = a[...].astype(jnp.float32)
    cv = c[...]
    cnt = cv[0][:, :1] + cv[1][:, :1]
    mean = (av[0] + av[1]) / jnp.maximum(cnt, 1.0)
    h = (jnp.maximum(mean + z1[...], 0.0)).astype(jnp.bfloat16)
    y2[...] = jnp.dot(
        h, wl[...], preferred_element_type=jnp.float32
    ).astype(jnp.bfloat16)
    z2[...] = jnp.dot(h, wr[...], preferred_element_type=jnp.float32) + b[...]


def _tc_c_body(a, c, z2, wo, b, o):
    av = a[...].astype(jnp.float32)
    cv = c[...]
    cnt = cv[0][:, :1] + cv[1][:, :1]
    mean = (av[0] + av[1]) / jnp.maximum(cnt, 1.0)
    h = jnp.maximum(mean + z2[...], 0.0)
    o[...] = jnp.sum(h * wo[...], axis=1) + b[0, 0]


def _rows_spec(w):
    return pl.BlockSpec((_BM, w), lambda i: (i, 0))


def _prows_spec(w):
    return pl.BlockSpec((_NC, _BM, w), lambda i: (0, i, 0))


def _full_spec(shape):
    return pl.BlockSpec(shape, lambda i: tuple(0 for _ in shape))


_GRID = (_NPAD // _BM,)

_tc_a = pl.pallas_call(
    _tc_a_body,
    grid=_GRID,
    in_specs=[_rows_spec(_D), _full_spec((_D, _H)), _full_spec((_D, _H)),
              _full_spec((1, _H))],
    out_specs=[_rows_spec(_H), _rows_spec(_H)],
    out_shape=[jax.ShapeDtypeStruct((_NPAD, _H), jnp.bfloat16),
               jax.ShapeDtypeStruct((_NPAD, _H), jnp.float32)],
)

_tc_b = pl.pallas_call(
    _tc_b_body,
    grid=_GRID,
    in_specs=[_prows_spec(_H), _prows_spec(16),
              _rows_spec(_H), _full_spec((_H, _H)), _full_spec((_H, _H)),
              _full_spec((1, _H))],
    out_specs=[_rows_spec(_H), _rows_spec(_H)],
    out_shape=[jax.ShapeDtypeStruct((_NPAD, _H), jnp.bfloat16),
               jax.ShapeDtypeStruct((_NPAD, _H), jnp.float32)],
)

_tc_c = pl.pallas_call(
    _tc_c_body,
    grid=_GRID,
    in_specs=[_prows_spec(_H), _prows_spec(16),
              _rows_spec(_H), _full_spec((1, _H)), _full_spec((1, 1))],
    out_specs=pl.BlockSpec((_BM,), lambda i: (i,)),
    out_shape=jax.ShapeDtypeStruct((_NPAD,), jnp.float32),
)


def kernel(x, edge_index, W1l, b1l, W1r, W2l, b2l, W2r, Wo, bo):
    xp = jnp.zeros((_NPAD, _D), jnp.float32).at[:_N].set(x)
    pad = jnp.full((_EPAD - _E,), _NPAD - 1, jnp.int32)
    srcm = jnp.concatenate([edge_index[0], pad]).reshape(_EPAD // _CHUNK, _CHUNK)
    dstm = jnp.concatenate([edge_index[1], pad]).reshape(_EPAD // _CHUNK, _CHUNK)
    za = jnp.zeros((_ROWS_PER_SUB, _H), jnp.bfloat16)
    zc = jnp.zeros((_ROWS_PER_SUB, 16), jnp.float32)
    ones_h = jnp.ones((_CHUNK, 16), jnp.float32)

    y1, z1 = _tc_a(xp, W1l, W1r, b1l.reshape(1, _H))
    agg1, cnt = _sc_pass_cnt(y1, srcm, dstm, za, zc, ones_h)
    y2, z2 = _tc_b(agg1, cnt, z1, W2l, W2r, b2l.reshape(1, _H))
    (agg2,) = _sc_pass(y2, srcm, dstm, za)
    out = _tc_c(agg2, cnt, z2, Wo.reshape(1, _H), bo.reshape(1, 1))
    return out[:_N]
